# baseline (speedup 1.0000x reference)
import functools

import jax
import jax.numpy as jnp
from jax import lax
from jax.experimental import pallas as pl
from jax.experimental.pallas import tpu as pltpu

NEG_SLOPE = 0.1
NUM_CLASSES = 5


def _proj_body(ifeat_ref, ufeat_ref, cjm_ref, cju_ref, wrev_ref, wfwd_ref,
               pm_ref, pu_ref):
    fm = (ifeat_ref[...] * cjm_ref[...]).astype(jnp.bfloat16)
    fu = (ufeat_ref[...] * cju_ref[...]).astype(jnp.bfloat16)
    pm_ref[0] = jnp.dot(fm, wrev_ref[0],
                        preferred_element_type=jnp.float32).astype(jnp.bfloat16)
    pu_ref[0] = jnp.dot(fu, wfwd_ref[0],
                        preferred_element_type=jnp.float32).astype(jnp.bfloat16)


def _project(ifeat, ufeat, cj_m, cj_u, w_rev, w_fwd):
    r_dim, k_m, d = w_rev.shape[0], w_rev.shape[1], w_rev.shape[2]
    k_u = w_fwd.shape[1]
    nm, nu = ifeat.shape[0], ufeat.shape[0]
    return pl.pallas_call(
        _proj_body,
        out_shape=(jax.ShapeDtypeStruct((r_dim, nm, d), jnp.bfloat16),
                   jax.ShapeDtypeStruct((r_dim, nu, d), jnp.bfloat16)),
        grid=(r_dim,),
        in_specs=[
            pl.BlockSpec((nm, k_m), lambda r: (0, 0)),
            pl.BlockSpec((nu, k_u), lambda r: (0, 0)),
            pl.BlockSpec((nm, 1), lambda r: (0, 0)),
            pl.BlockSpec((nu, 1), lambda r: (0, 0)),
            pl.BlockSpec((1, k_m, d), lambda r: (r, 0, 0)),
            pl.BlockSpec((1, k_u, d), lambda r: (r, 0, 0)),
        ],
        out_specs=(pl.BlockSpec((1, nm, d), lambda r: (r, 0, 0)),
                   pl.BlockSpec((1, nu, d), lambda r: (r, 0, 0))),
        compiler_params=pltpu.CompilerParams(
            dimension_semantics=("parallel",),
            vmem_limit_bytes=60 * 1024 * 1024),
    )(ifeat, ufeat, cj_m, cj_u, w_rev.astype(jnp.bfloat16),
      w_fwd.astype(jnp.bfloat16))


def _enc_body(a_ref, proj_ref, ci_ref, fcw_ref, fcb_ref, out_ref,
              *, r_dim, transpose_a):
    acc = None
    for r in range(r_dim):
        a = a_ref[r]
        p = proj_ref[r]
        if transpose_a:
            part = lax.dot_general(a, p, (((0,), (0,)), ((), ())),
                                   preferred_element_type=jnp.float32)
        else:
            part = jnp.dot(a, p, preferred_element_type=jnp.float32)
        acc = part if acc is None else acc + part
    h = acc * ci_ref[...]
    h = jnp.where(h > 0, h, NEG_SLOPE * h)
    y = jnp.dot(h.astype(jnp.bfloat16), fcw_ref[...],
                preferred_element_type=jnp.float32) + fcb_ref[...]
    out_ref[...] = y


def _encode(a_stack, proj, ci, fc_w, fc_b, *, transpose_a, tile_m):
    r_dim = a_stack.shape[0]
    if transpose_a:
        nsrc, ndst = a_stack.shape[1], a_stack.shape[2]
    else:
        ndst, nsrc = a_stack.shape[1], a_stack.shape[2]
    d = proj.shape[2]
    dout = fc_w.shape[1]
    tm = min(tile_m, ndst)
    assert ndst % tm == 0 and proj.shape[1] == nsrc

    if transpose_a:
        a_spec = pl.BlockSpec((r_dim, nsrc, tm), lambda i: (0, 0, i))
    else:
        a_spec = pl.BlockSpec((r_dim, tm, nsrc), lambda i: (0, i, 0))

    body = functools.partial(_enc_body, r_dim=r_dim, transpose_a=transpose_a)
    return pl.pallas_call(
        body,
        out_shape=jax.ShapeDtypeStruct((ndst, dout), jnp.float32),
        grid=(ndst // tm,),
        in_specs=[
            a_spec,
            pl.BlockSpec((r_dim, nsrc, d), lambda i: (0, 0, 0)),
            pl.BlockSpec((tm, 1), lambda i: (i, 0)),
            pl.BlockSpec((d, dout), lambda i: (0, 0)),
            pl.BlockSpec((1, dout), lambda i: (0, 0)),
        ],
        out_specs=pl.BlockSpec((tm, dout), lambda i: (i, 0)),
        compiler_params=pltpu.CompilerParams(
            dimension_semantics=("parallel",),
            vmem_limit_bytes=60 * 1024 * 1024),
    )(a_stack, proj, ci, fc_w.astype(jnp.bfloat16), fc_b)


def _dec_body(h_ref, t_ref, uu_ref, vv_ref, pw_ref, wcat_ref, out_ref,
              scru, scrv, *, tile_e, unroll):
    d = uu_ref.shape[1]
    nb = pw_ref.shape[1] // d
    nchunk = tile_e // unroll

    def step(ci, carry):
        cbase = pl.multiple_of(jnp.maximum(ci - 1, 0) * unroll, unroll)
        u = scru[pl.ds(cbase, unroll), :].astype(jnp.bfloat16)
        t = jnp.dot(u, pw_ref[...],
                    preferred_element_type=jnp.float32)
        v = scrv[pl.ds(cbase, unroll), :]
        vv = jnp.concatenate([v] * nb, axis=1)
        pc = (t * vv).astype(jnp.bfloat16)
        out_ref[pl.ds(cbase, unroll), :] = jnp.dot(
            pc, wcat_ref[...], preferred_element_type=jnp.float32)

        gbase = jnp.minimum(ci, nchunk - 1) * unroll
        for k in range(unroll):
            e = gbase + k
            hi = pl.multiple_of(h_ref[0, 0, e], 2)
            scru[pl.ds(e, 1), :] = uu_ref[pl.ds(hi, 2), :][0:1, :]
            ti = pl.multiple_of(t_ref[0, 0, e], 2)
            scrv[pl.ds(e, 1), :] = vv_ref[pl.ds(ti, 2), :][0:1, :]
        return carry

    lax.fori_loop(0, nchunk + 1, step, 0)


def _decode(u_tab, v_tab, h2, t2, p_wide, wcat, *, tile_e=4096, unroll=128):
    n2, d = u_tab.shape
    c_p = wcat.shape[1]
    e_p = h2.shape[0] * h2.shape[2]
    te = min(tile_e, e_p)
    assert e_p % te == 0 and h2.shape[2] == te
    body = functools.partial(_dec_body, tile_e=te, unroll=unroll)
    return pl.pallas_call(
        body,
        out_shape=jax.ShapeDtypeStruct((e_p, c_p), jnp.float32),
        grid=(e_p // te,),
        in_specs=[
            pl.BlockSpec((1, 1, te), lambda e: (e, 0, 0),
                         memory_space=pltpu.SMEM),
            pl.BlockSpec((1, 1, te), lambda e: (e, 0, 0),
                         memory_space=pltpu.SMEM),
            pl.BlockSpec((n2, d), lambda e: (0, 0)),
            pl.BlockSpec((n2, d), lambda e: (0, 0)),
            pl.BlockSpec(p_wide.shape, lambda e: (0, 0)),
            pl.BlockSpec(wcat.shape, lambda e: (0, 0)),
        ],
        out_specs=pl.BlockSpec((te, c_p), lambda e: (e, 0)),
        scratch_shapes=[pltpu.VMEM((te, d), jnp.float32),
                        pltpu.VMEM((te, d), jnp.float32)],
        compiler_params=pltpu.CompilerParams(
            dimension_semantics=("parallel",),
            vmem_limit_bytes=60 * 1024 * 1024),
    )(h2, t2, u_tab, v_tab, p_wide, wcat)


def kernel(a_stack, ufeat, ifeat, cj_u, ci_u, cj_m, ci_m,
           w_fwd, w_rev, ufc_w, ufc_b, ifc_w, ifc_b,
           wc, p_wide, head_idx, tail_idx):
    proj_m, proj_u = _project(ifeat, ufeat, cj_m, cj_u, w_rev, w_fwd)

    user_out = _encode(a_stack, proj_m, ci_u, ufc_w, ufc_b,
                       transpose_a=False, tile_m=512)
    movie_out = _encode(a_stack, proj_u, ci_m, ifc_w, ifc_b,
                        transpose_a=True, tile_m=512)

    dout = ufc_w.shape[1]
    u_tab = jnp.repeat(user_out, 2, axis=0)
    v_tab = jnp.repeat(movie_out, 2, axis=0)

    e_p = head_idx.shape[0]
    te = min(4096, e_p)
    h2 = (head_idx * 2).astype(jnp.int32).reshape(e_p // te, 1, te)
    t2 = (tail_idx * 2).astype(jnp.int32).reshape(e_p // te, 1, te)

    nb, c_p = wc.shape
    wcat = jnp.concatenate(
        [jnp.broadcast_to(wc[b:b + 1, :], (dout, c_p)) for b in range(nb)],
        axis=0).astype(jnp.bfloat16)

    pred = _decode(u_tab, v_tab, h2, t2, p_wide.astype(jnp.bfloat16), wcat)
    return pred[:e_p, :NUM_CLASSES]


# speedup vs baseline: 1.2039x; 1.2039x over previous
import functools

import jax
import jax.numpy as jnp
from jax import lax
from jax.experimental import pallas as pl
from jax.experimental.pallas import tpu as pltpu

NEG_SLOPE = 0.1
NUM_CLASSES = 5


def _proj_body(ifeat_ref, ufeat_ref, cjm_ref, cju_ref, wrev_ref, wfwd_ref,
               pm_ref, pu_ref):
    fm = (ifeat_ref[...] * cjm_ref[...]).astype(jnp.bfloat16)
    fu = (ufeat_ref[...] * cju_ref[...]).astype(jnp.bfloat16)
    pm_ref[0] = jnp.dot(fm, wrev_ref[0],
                        preferred_element_type=jnp.float32).astype(jnp.bfloat16)
    pu_ref[0] = jnp.dot(fu, wfwd_ref[0],
                        preferred_element_type=jnp.float32).astype(jnp.bfloat16)


def _project(ifeat, ufeat, cj_m, cj_u, w_rev, w_fwd):
    r_dim, k_m, d = w_rev.shape[0], w_rev.shape[1], w_rev.shape[2]
    k_u = w_fwd.shape[1]
    nm, nu = ifeat.shape[0], ufeat.shape[0]
    return pl.pallas_call(
        _proj_body,
        out_shape=(jax.ShapeDtypeStruct((r_dim, nm, d), jnp.bfloat16),
                   jax.ShapeDtypeStruct((r_dim, nu, d), jnp.bfloat16)),
        grid=(r_dim,),
        in_specs=[
            pl.BlockSpec((nm, k_m), lambda r: (0, 0)),
            pl.BlockSpec((nu, k_u), lambda r: (0, 0)),
            pl.BlockSpec((nm, 1), lambda r: (0, 0)),
            pl.BlockSpec((nu, 1), lambda r: (0, 0)),
            pl.BlockSpec((1, k_m, d), lambda r: (r, 0, 0)),
            pl.BlockSpec((1, k_u, d), lambda r: (r, 0, 0)),
        ],
        out_specs=(pl.BlockSpec((1, nm, d), lambda r: (r, 0, 0)),
                   pl.BlockSpec((1, nu, d), lambda r: (r, 0, 0))),
        compiler_params=pltpu.CompilerParams(
            dimension_semantics=("parallel",),
            vmem_limit_bytes=60 * 1024 * 1024),
    )(ifeat, ufeat, cj_m, cj_u, w_rev.astype(jnp.bfloat16),
      w_fwd.astype(jnp.bfloat16))


def _enc_body(a_ref, proj_ref, ci_ref, fcw_ref, fcb_ref, out_ref,
              *, r_dim, transpose_a):
    acc = None
    for r in range(r_dim):
        a = a_ref[r]
        p = proj_ref[r]
        if transpose_a:
            part = lax.dot_general(a, p, (((0,), (0,)), ((), ())),
                                   preferred_element_type=jnp.float32)
        else:
            part = jnp.dot(a, p, preferred_element_type=jnp.float32)
        acc = part if acc is None else acc + part
    h = acc * ci_ref[...]
    h = jnp.where(h > 0, h, NEG_SLOPE * h)
    y = jnp.dot(h.astype(jnp.bfloat16), fcw_ref[...],
                preferred_element_type=jnp.float32) + fcb_ref[...]
    out_ref[...] = y


def _encode(a_stack, proj, ci, fc_w, fc_b, *, transpose_a, tile_m):
    r_dim = a_stack.shape[0]
    if transpose_a:
        nsrc, ndst = a_stack.shape[1], a_stack.shape[2]
    else:
        ndst, nsrc = a_stack.shape[1], a_stack.shape[2]
    d = proj.shape[2]
    dout = fc_w.shape[1]
    tm = min(tile_m, ndst)
    assert ndst % tm == 0 and proj.shape[1] == nsrc

    if transpose_a:
        a_spec = pl.BlockSpec((r_dim, nsrc, tm), lambda i: (0, 0, i))
    else:
        a_spec = pl.BlockSpec((r_dim, tm, nsrc), lambda i: (0, i, 0))

    body = functools.partial(_enc_body, r_dim=r_dim, transpose_a=transpose_a)
    return pl.pallas_call(
        body,
        out_shape=jax.ShapeDtypeStruct((ndst, dout), jnp.float32),
        grid=(ndst // tm,),
        in_specs=[
            a_spec,
            pl.BlockSpec((r_dim, nsrc, d), lambda i: (0, 0, 0)),
            pl.BlockSpec((tm, 1), lambda i: (i, 0)),
            pl.BlockSpec((d, dout), lambda i: (0, 0)),
            pl.BlockSpec((1, dout), lambda i: (0, 0)),
        ],
        out_specs=pl.BlockSpec((tm, dout), lambda i: (i, 0)),
        compiler_params=pltpu.CompilerParams(
            dimension_semantics=("parallel",),
            vmem_limit_bytes=60 * 1024 * 1024),
    )(a_stack, proj, ci, fc_w.astype(jnp.bfloat16), fc_b)


def _dec_body(h_ref, t_ref, uu_ref, vv_ref, pw_ref, wcat_ref, out_ref,
              scru, scrv, *, tile_e, unroll):
    def chunk(ci, carry):
        base = ci * unroll
        for k in range(unroll):
            e = base + k
            hi = pl.multiple_of(h_ref[0, 0, e], 2)
            scru[pl.ds(e, 1), :] = uu_ref[pl.ds(hi, 2), :][0:1, :]
            ti = pl.multiple_of(t_ref[0, 0, e], 2)
            scrv[pl.ds(e, 1), :] = vv_ref[pl.ds(ti, 2), :][0:1, :]
        return carry

    lax.fori_loop(0, tile_e // unroll, chunk, 0)

    u = scru[...].astype(jnp.bfloat16)
    t = jnp.dot(u, pw_ref[...],
                preferred_element_type=jnp.float32)
    v = scrv[...]
    nb = pw_ref.shape[1] // v.shape[1]
    vv = jnp.concatenate([v] * nb, axis=1)
    pc = (t * vv).astype(jnp.bfloat16)
    out_ref[...] = jnp.dot(pc, wcat_ref[...],
                           preferred_element_type=jnp.float32)


def _decode(u_tab, v_tab, h2, t2, p_wide, wcat, *, tile_e=4096, unroll=256):
    n2, d = u_tab.shape
    c_p = wcat.shape[1]
    e_p = h2.shape[0] * h2.shape[2]
    te = min(tile_e, e_p)
    assert e_p % te == 0 and h2.shape[2] == te
    body = functools.partial(_dec_body, tile_e=te, unroll=unroll)
    return pl.pallas_call(
        body,
        out_shape=jax.ShapeDtypeStruct((e_p, c_p), jnp.float32),
        grid=(e_p // te,),
        in_specs=[
            pl.BlockSpec((1, 1, te), lambda e: (e, 0, 0),
                         memory_space=pltpu.SMEM),
            pl.BlockSpec((1, 1, te), lambda e: (e, 0, 0),
                         memory_space=pltpu.SMEM),
            pl.BlockSpec((n2, d), lambda e: (0, 0)),
            pl.BlockSpec((n2, d), lambda e: (0, 0)),
            pl.BlockSpec(p_wide.shape, lambda e: (0, 0)),
            pl.BlockSpec(wcat.shape, lambda e: (0, 0)),
        ],
        out_specs=pl.BlockSpec((te, c_p), lambda e: (e, 0)),
        scratch_shapes=[pltpu.VMEM((te, d), jnp.float32),
                        pltpu.VMEM((te, d), jnp.float32)],
        compiler_params=pltpu.CompilerParams(
            dimension_semantics=("parallel",),
            vmem_limit_bytes=60 * 1024 * 1024),
    )(h2, t2, u_tab, v_tab, p_wide, wcat)


def kernel(a_stack, ufeat, ifeat, cj_u, ci_u, cj_m, ci_m,
           w_fwd, w_rev, ufc_w, ufc_b, ifc_w, ifc_b,
           wc, p_wide, head_idx, tail_idx):
    proj_m, proj_u = _project(ifeat, ufeat, cj_m, cj_u, w_rev, w_fwd)

    user_out = _encode(a_stack, proj_m, ci_u, ufc_w, ufc_b,
                       transpose_a=False, tile_m=512)
    movie_out = _encode(a_stack, proj_u, ci_m, ifc_w, ifc_b,
                        transpose_a=True, tile_m=512)

    dout = ufc_w.shape[1]
    u_tab = jnp.repeat(user_out, 2, axis=0)
    v_tab = jnp.repeat(movie_out, 2, axis=0)

    e_p = head_idx.shape[0]
    te = min(4096, e_p)
    h2 = (head_idx * 2).astype(jnp.int32).reshape(e_p // te, 1, te)
    t2 = (tail_idx * 2).astype(jnp.int32).reshape(e_p // te, 1, te)

    nb, c_p = wc.shape
    wcat = jnp.concatenate(
        [jnp.broadcast_to(wc[b:b + 1, :], (dout, c_p)) for b in range(nb)],
        axis=0).astype(jnp.bfloat16)

    pred = _decode(u_tab, v_tab, h2, t2, p_wide.astype(jnp.bfloat16), wcat)
    return pred[:e_p, :NUM_CLASSES]


# speedup vs baseline: 1.2089x; 1.0042x over previous
import functools

import jax
import jax.numpy as jnp
from jax import lax
from jax.experimental import pallas as pl
from jax.experimental.pallas import tpu as pltpu

NEG_SLOPE = 0.1
NUM_CLASSES = 5


def _proj_body(ifeat_ref, ufeat_ref, cjm_ref, cju_ref, wrev_ref, wfwd_ref,
               pm_ref, pu_ref):
    fm = (ifeat_ref[...] * cjm_ref[...]).astype(jnp.bfloat16)
    fu = (ufeat_ref[...] * cju_ref[...]).astype(jnp.bfloat16)
    pm_ref[0] = jnp.dot(fm, wrev_ref[0],
                        preferred_element_type=jnp.float32).astype(jnp.bfloat16)
    pu_ref[0] = jnp.dot(fu, wfwd_ref[0],
                        preferred_element_type=jnp.float32).astype(jnp.bfloat16)


def _project(ifeat, ufeat, cj_m, cj_u, w_rev, w_fwd):
    r_dim, k_m, d = w_rev.shape[0], w_rev.shape[1], w_rev.shape[2]
    k_u = w_fwd.shape[1]
    nm, nu = ifeat.shape[0], ufeat.shape[0]
    return pl.pallas_call(
        _proj_body,
        out_shape=(jax.ShapeDtypeStruct((r_dim, nm, d), jnp.bfloat16),
                   jax.ShapeDtypeStruct((r_dim, nu, d), jnp.bfloat16)),
        grid=(r_dim,),
        in_specs=[
            pl.BlockSpec((nm, k_m), lambda r: (0, 0)),
            pl.BlockSpec((nu, k_u), lambda r: (0, 0)),
            pl.BlockSpec((nm, 1), lambda r: (0, 0)),
            pl.BlockSpec((nu, 1), lambda r: (0, 0)),
            pl.BlockSpec((1, k_m, d), lambda r: (r, 0, 0)),
            pl.BlockSpec((1, k_u, d), lambda r: (r, 0, 0)),
        ],
        out_specs=(pl.BlockSpec((1, nm, d), lambda r: (r, 0, 0)),
                   pl.BlockSpec((1, nu, d), lambda r: (r, 0, 0))),
        compiler_params=pltpu.CompilerParams(
            dimension_semantics=("parallel",),
            vmem_limit_bytes=60 * 1024 * 1024),
    )(ifeat, ufeat, cj_m, cj_u, w_rev.astype(jnp.bfloat16),
      w_fwd.astype(jnp.bfloat16))


def _enc_body(a_ref, proj_ref, ci_ref, fcw_ref, fcb_ref, out_ref,
              *, r_dim, transpose_a):
    acc = None
    for r in range(r_dim):
        a = a_ref[r]
        p = proj_ref[r]
        if transpose_a:
            part = lax.dot_general(a, p, (((0,), (0,)), ((), ())),
                                   preferred_element_type=jnp.float32)
        else:
            part = jnp.dot(a, p, preferred_element_type=jnp.float32)
        acc = part if acc is None else acc + part
    h = acc * ci_ref[...]
    h = jnp.where(h > 0, h, NEG_SLOPE * h)
    y = jnp.dot(h.astype(jnp.bfloat16), fcw_ref[...],
                preferred_element_type=jnp.float32) + fcb_ref[...]
    out_ref[...] = y


def _encode(a_stack, proj, ci, fc_w, fc_b, *, transpose_a, tile_m):
    r_dim = a_stack.shape[0]
    if transpose_a:
        nsrc, ndst = a_stack.shape[1], a_stack.shape[2]
    else:
        ndst, nsrc = a_stack.shape[1], a_stack.shape[2]
    d = proj.shape[2]
    dout = fc_w.shape[1]
    tm = min(tile_m, ndst)
    assert ndst % tm == 0 and proj.shape[1] == nsrc

    if transpose_a:
        a_spec = pl.BlockSpec((r_dim, nsrc, tm), lambda i: (0, 0, i))
    else:
        a_spec = pl.BlockSpec((r_dim, tm, nsrc), lambda i: (0, i, 0))

    body = functools.partial(_enc_body, r_dim=r_dim, transpose_a=transpose_a)
    return pl.pallas_call(
        body,
        out_shape=jax.ShapeDtypeStruct((ndst, dout), jnp.float32),
        grid=(ndst // tm,),
        in_specs=[
            a_spec,
            pl.BlockSpec((r_dim, nsrc, d), lambda i: (0, 0, 0)),
            pl.BlockSpec((tm, 1), lambda i: (i, 0)),
            pl.BlockSpec((d, dout), lambda i: (0, 0)),
            pl.BlockSpec((1, dout), lambda i: (0, 0)),
        ],
        out_specs=pl.BlockSpec((tm, dout), lambda i: (i, 0)),
        compiler_params=pltpu.CompilerParams(
            dimension_semantics=("parallel",),
            vmem_limit_bytes=60 * 1024 * 1024),
    )(a_stack, proj, ci, fc_w.astype(jnp.bfloat16), fc_b)


def _dec_body(h_ref, t_ref, uu_ref, vv_ref, pw_ref, wcat_ref, out_ref,
              scru, scrv, *, tile_e, unroll):
    def chunk(ci, carry):
        base = ci * unroll
        for k in range(unroll):
            e = base + k
            hi = pl.multiple_of(h_ref[0, 0, e], 2)
            scru[pl.ds(e, 1), :] = uu_ref[pl.ds(hi, 2), :][0:1, :]
            ti = pl.multiple_of(t_ref[0, 0, e], 2)
            scrv[pl.ds(e, 1), :] = vv_ref[pl.ds(ti, 2), :][0:1, :]
        return carry

    lax.fori_loop(0, tile_e // unroll, chunk, 0)

    u = scru[...].astype(jnp.bfloat16)
    t = jnp.dot(u, pw_ref[...],
                preferred_element_type=jnp.float32)
    v = scrv[...]
    nb = pw_ref.shape[1] // v.shape[1]
    vv = jnp.concatenate([v] * nb, axis=1)
    pc = (t * vv).astype(jnp.bfloat16)
    out_ref[...] = jnp.dot(pc, wcat_ref[...],
                           preferred_element_type=jnp.float32)


def _decode(u_tab, v_tab, h2, t2, p_wide, wcat, *, tile_e=5120, unroll=256):
    n2, d = u_tab.shape
    c_p = wcat.shape[1]
    e_p = h2.shape[0] * h2.shape[2]
    te = min(tile_e, e_p)
    assert e_p % te == 0 and h2.shape[2] == te
    body = functools.partial(_dec_body, tile_e=te, unroll=unroll)
    return pl.pallas_call(
        body,
        out_shape=jax.ShapeDtypeStruct((e_p, c_p), jnp.float32),
        grid=(e_p // te,),
        in_specs=[
            pl.BlockSpec((1, 1, te), lambda e: (e, 0, 0),
                         memory_space=pltpu.SMEM),
            pl.BlockSpec((1, 1, te), lambda e: (e, 0, 0),
                         memory_space=pltpu.SMEM),
            pl.BlockSpec((n2, d), lambda e: (0, 0)),
            pl.BlockSpec((n2, d), lambda e: (0, 0)),
            pl.BlockSpec(p_wide.shape, lambda e: (0, 0)),
            pl.BlockSpec(wcat.shape, lambda e: (0, 0)),
        ],
        out_specs=pl.BlockSpec((te, c_p), lambda e: (e, 0)),
        scratch_shapes=[pltpu.VMEM((te, d), jnp.float32),
                        pltpu.VMEM((te, d), jnp.float32)],
        compiler_params=pltpu.CompilerParams(
            dimension_semantics=("parallel",),
            vmem_limit_bytes=60 * 1024 * 1024),
    )(h2, t2, u_tab, v_tab, p_wide, wcat)


def kernel(a_stack, ufeat, ifeat, cj_u, ci_u, cj_m, ci_m,
           w_fwd, w_rev, ufc_w, ufc_b, ifc_w, ifc_b,
           wc, p_wide, head_idx, tail_idx):
    proj_m, proj_u = _project(ifeat, ufeat, cj_m, cj_u, w_rev, w_fwd)

    user_out = _encode(a_stack, proj_m, ci_u, ufc_w, ufc_b,
                       transpose_a=False, tile_m=512)
    movie_out = _encode(a_stack, proj_u, ci_m, ifc_w, ifc_b,
                        transpose_a=True, tile_m=512)

    dout = ufc_w.shape[1]
    u_tab = jnp.repeat(user_out, 2, axis=0)
    v_tab = jnp.repeat(movie_out, 2, axis=0)

    e_p = head_idx.shape[0]
    te = min(5120, e_p)
    h2 = (head_idx * 2).astype(jnp.int32).reshape(e_p // te, 1, te)
    t2 = (tail_idx * 2).astype(jnp.int32).reshape(e_p // te, 1, te)

    nb, c_p = wc.shape
    wcat = jnp.concatenate(
        [jnp.broadcast_to(wc[b:b + 1, :], (dout, c_p)) for b in range(nb)],
        axis=0).astype(jnp.bfloat16)

    pred = _decode(u_tab, v_tab, h2, t2, p_wide.astype(jnp.bfloat16), wcat)
    return pred[:e_p, :NUM_CLASSES]


# speedup vs baseline: 1.2128x; 1.0032x over previous
import functools

import jax
import jax.numpy as jnp
from jax import lax
from jax.experimental import pallas as pl
from jax.experimental.pallas import tpu as pltpu

NEG_SLOPE = 0.1
NUM_CLASSES = 5


def _proj_body(ifeat_ref, ufeat_ref, cjm_ref, cju_ref, wrev_ref, wfwd_ref,
               pm_ref, pu_ref):
    fm = (ifeat_ref[...] * cjm_ref[...]).astype(jnp.bfloat16)
    fu = (ufeat_ref[...] * cju_ref[...]).astype(jnp.bfloat16)
    pm_ref[0] = jnp.dot(fm, wrev_ref[0],
                        preferred_element_type=jnp.float32).astype(jnp.bfloat16)
    pu_ref[0] = jnp.dot(fu, wfwd_ref[0],
                        preferred_element_type=jnp.float32).astype(jnp.bfloat16)


def _project(ifeat, ufeat, cj_m, cj_u, w_rev, w_fwd):
    r_dim, k_m, d = w_rev.shape[0], w_rev.shape[1], w_rev.shape[2]
    k_u = w_fwd.shape[1]
    nm, nu = ifeat.shape[0], ufeat.shape[0]
    return pl.pallas_call(
        _proj_body,
        out_shape=(jax.ShapeDtypeStruct((r_dim, nm, d), jnp.bfloat16),
                   jax.ShapeDtypeStruct((r_dim, nu, d), jnp.bfloat16)),
        grid=(r_dim,),
        in_specs=[
            pl.BlockSpec((nm, k_m), lambda r: (0, 0)),
            pl.BlockSpec((nu, k_u), lambda r: (0, 0)),
            pl.BlockSpec((nm, 1), lambda r: (0, 0)),
            pl.BlockSpec((nu, 1), lambda r: (0, 0)),
            pl.BlockSpec((1, k_m, d), lambda r: (r, 0, 0)),
            pl.BlockSpec((1, k_u, d), lambda r: (r, 0, 0)),
        ],
        out_specs=(pl.BlockSpec((1, nm, d), lambda r: (r, 0, 0)),
                   pl.BlockSpec((1, nu, d), lambda r: (r, 0, 0))),
        compiler_params=pltpu.CompilerParams(
            dimension_semantics=("parallel",),
            vmem_limit_bytes=60 * 1024 * 1024),
    )(ifeat, ufeat, cj_m, cj_u, w_rev.astype(jnp.bfloat16),
      w_fwd.astype(jnp.bfloat16))


def _enc_body(a_ref, proj_ref, ci_ref, fcw_ref, fcb_ref, out_ref,
              *, r_dim, transpose_a):
    acc = None
    for r in range(r_dim):
        a = a_ref[r]
        p = proj_ref[r]
        if transpose_a:
            part = lax.dot_general(a, p, (((0,), (0,)), ((), ())),
                                   preferred_element_type=jnp.float32)
        else:
            part = jnp.dot(a, p, preferred_element_type=jnp.float32)
        acc = part if acc is None else acc + part
    h = acc * ci_ref[...]
    h = jnp.where(h > 0, h, NEG_SLOPE * h)
    y = jnp.dot(h.astype(jnp.bfloat16), fcw_ref[...],
                preferred_element_type=jnp.float32) + fcb_ref[...]
    out_ref[...] = y


def _encode(a_stack, proj, ci, fc_w, fc_b, *, transpose_a, tile_m):
    r_dim = a_stack.shape[0]
    if transpose_a:
        nsrc, ndst = a_stack.shape[1], a_stack.shape[2]
    else:
        ndst, nsrc = a_stack.shape[1], a_stack.shape[2]
    d = proj.shape[2]
    dout = fc_w.shape[1]
    tm = min(tile_m, ndst)
    assert ndst % tm == 0 and proj.shape[1] == nsrc

    if transpose_a:
        a_spec = pl.BlockSpec((r_dim, nsrc, tm), lambda i: (0, 0, i))
    else:
        a_spec = pl.BlockSpec((r_dim, tm, nsrc), lambda i: (0, i, 0))

    body = functools.partial(_enc_body, r_dim=r_dim, transpose_a=transpose_a)
    return pl.pallas_call(
        body,
        out_shape=jax.ShapeDtypeStruct((ndst, dout), jnp.float32),
        grid=(ndst // tm,),
        in_specs=[
            a_spec,
            pl.BlockSpec((r_dim, nsrc, d), lambda i: (0, 0, 0)),
            pl.BlockSpec((tm, 1), lambda i: (i, 0)),
            pl.BlockSpec((d, dout), lambda i: (0, 0)),
            pl.BlockSpec((1, dout), lambda i: (0, 0)),
        ],
        out_specs=pl.BlockSpec((tm, dout), lambda i: (i, 0)),
        compiler_params=pltpu.CompilerParams(
            dimension_semantics=("parallel",),
            vmem_limit_bytes=60 * 1024 * 1024),
    )(a_stack, proj, ci, fc_w.astype(jnp.bfloat16), fc_b)


def _dec_body(h_ref, t_ref, uu_ref, vv_ref, pw_ref, wcat_ref, out_ref,
              scru, scrv, *, tile_e, unroll):
    def chunk(ci, carry):
        base = ci * unroll
        for k in range(unroll):
            e = base + k
            hi = pl.multiple_of(h_ref[0, 0, e], 2)
            scru[pl.ds(e, 1), :] = uu_ref[pl.ds(hi, 2), :][0:1, :]
            ti = pl.multiple_of(t_ref[0, 0, e], 2)
            scrv[pl.ds(e, 1), :] = vv_ref[pl.ds(ti, 2), :][0:1, :]
        return carry

    lax.fori_loop(0, tile_e // unroll, chunk, 0)

    u = scru[...].astype(jnp.bfloat16)
    t = jnp.dot(u, pw_ref[...],
                preferred_element_type=jnp.float32)
    v = scrv[...]
    nb = pw_ref.shape[1] // v.shape[1]
    vv = jnp.concatenate([v] * nb, axis=1)
    pc = (t * vv).astype(jnp.bfloat16)
    out_ref[...] = jnp.dot(pc, wcat_ref[...],
                           preferred_element_type=jnp.float32)


def _decode(u_tab, v_tab, h2, t2, p_wide, wcat, *, tile_e=10240, unroll=256):
    n2, d = u_tab.shape
    c_p = wcat.shape[1]
    e_p = h2.shape[0] * h2.shape[2]
    te = min(tile_e, e_p)
    assert e_p % te == 0 and h2.shape[2] == te
    body = functools.partial(_dec_body, tile_e=te, unroll=unroll)
    return pl.pallas_call(
        body,
        out_shape=jax.ShapeDtypeStruct((e_p, c_p), jnp.float32),
        grid=(e_p // te,),
        in_specs=[
            pl.BlockSpec((1, 1, te), lambda e: (e, 0, 0),
                         memory_space=pltpu.SMEM),
            pl.BlockSpec((1, 1, te), lambda e: (e, 0, 0),
                         memory_space=pltpu.SMEM),
            pl.BlockSpec((n2, d), lambda e: (0, 0)),
            pl.BlockSpec((n2, d), lambda e: (0, 0)),
            pl.BlockSpec(p_wide.shape, lambda e: (0, 0)),
            pl.BlockSpec(wcat.shape, lambda e: (0, 0)),
        ],
        out_specs=pl.BlockSpec((te, c_p), lambda e: (e, 0)),
        scratch_shapes=[pltpu.VMEM((te, d), jnp.float32),
                        pltpu.VMEM((te, d), jnp.float32)],
        compiler_params=pltpu.CompilerParams(
            dimension_semantics=("parallel",),
            vmem_limit_bytes=60 * 1024 * 1024),
    )(h2, t2, u_tab, v_tab, p_wide, wcat)


def kernel(a_stack, ufeat, ifeat, cj_u, ci_u, cj_m, ci_m,
           w_fwd, w_rev, ufc_w, ufc_b, ifc_w, ifc_b,
           wc, p_wide, head_idx, tail_idx):
    proj_m, proj_u = _project(ifeat, ufeat, cj_m, cj_u, w_rev, w_fwd)

    user_out = _encode(a_stack, proj_m, ci_u, ufc_w, ufc_b,
                       transpose_a=False, tile_m=512)
    movie_out = _encode(a_stack, proj_u, ci_m, ifc_w, ifc_b,
                        transpose_a=True, tile_m=512)

    dout = ufc_w.shape[1]
    u_tab = jnp.repeat(user_out, 2, axis=0)
    v_tab = jnp.repeat(movie_out, 2, axis=0)

    e_p = head_idx.shape[0]
    te = min(10240, e_p)
    h2 = (head_idx * 2).astype(jnp.int32).reshape(e_p // te, 1, te)
    t2 = (tail_idx * 2).astype(jnp.int32).reshape(e_p // te, 1, te)

    nb, c_p = wc.shape
    wcat = jnp.concatenate(
        [jnp.broadcast_to(wc[b:b + 1, :], (dout, c_p)) for b in range(nb)],
        axis=0).astype(jnp.bfloat16)

    pred = _decode(u_tab, v_tab, h2, t2, p_wide.astype(jnp.bfloat16), wcat)
    return pred[:e_p, :NUM_CLASSES]
